# Initial kernel scaffold; baseline (speedup 1.0000x reference)
import jax
import jax.numpy as jnp
from jax.experimental import pallas as pl


def kernel(x, edge_index, edge_weight, W1, b1, W2, b2, Wh1, bh1, Wh2, bh2, alpha, Wc, bc):
    raise NotImplementedError("write your pallas kernel here")


# speedup vs baseline: 2.6015x; 2.6015x over previous
import functools

import jax
import jax.numpy as jnp
from jax import lax
from jax.experimental import pallas as pl
from jax.experimental.pallas import tpu as pltpu
from jax.experimental.pallas import tpu_sc as plsc

N = 10000
E = 320000
D = 128
NCLS = 40
NC = 2
NS = 16
L = 16

EPT = 20480
EP = NS * EPT
K = 128
CH = EPT // K
RB = 200
NBLK = N // RB
NFULL = NBLK // NS
NTAIL = NBLK - NFULL * NS


def _mm_body(x_ref, w_ref, o_ref):
    o_ref[...] = jnp.dot(x_ref[...], w_ref[0], preferred_element_type=jnp.float32)


def _mm_relu_body(x_ref, b_ref, w_ref, o_ref):
    h = jnp.maximum(x_ref[...] + b_ref[0], 0.0)
    o_ref[...] = jnp.dot(h, w_ref[0], preferred_element_type=jnp.float32)


_RB_TC = 1000


def _dense_first(x, w_stacked):
    return pl.pallas_call(
        _mm_body,
        grid=(2 * N // _RB_TC,),
        in_specs=[
            pl.BlockSpec((_RB_TC, D), lambda i: (i % (N // _RB_TC), 0)),
            pl.BlockSpec((1, D, D), lambda i: (i // (N // _RB_TC), 0, 0)),
        ],
        out_specs=pl.BlockSpec((_RB_TC, D), lambda i: (i, 0)),
        out_shape=jax.ShapeDtypeStruct((2 * N, D), jnp.float32),
    )(x, w_stacked)


def _dense_mid(agg, b_stacked, w_stacked):
    return pl.pallas_call(
        _mm_relu_body,
        grid=(2 * N // _RB_TC,),
        in_specs=[
            pl.BlockSpec((_RB_TC, D), lambda i: (i, 0)),
            pl.BlockSpec((1, 1, D), lambda i: (i // (N // _RB_TC), 0, 0)),
            pl.BlockSpec((1, D, D), lambda i: (i // (N // _RB_TC), 0, 0)),
        ],
        out_specs=pl.BlockSpec((_RB_TC, D), lambda i: (i, 0)),
        out_shape=jax.ShapeDtypeStruct((2 * N, D), jnp.float32),
    )(agg, b_stacked, w_stacked)


def _final_body(g_ref, h_ref, b2_ref, bh2_ref, alpha_ref, wc_ref, bc_ref,
                zg_ref, zh_ref, z_ref, lg_ref):
    def l2n(v):
        nrm = jnp.sqrt(jnp.sum(v * v, axis=1, keepdims=True))
        return v / jnp.maximum(nrm, 1e-12)

    zg = l2n(g_ref[...] + b2_ref[...])
    zh = l2n(h_ref[...] + bh2_ref[...])
    a = alpha_ref[0, 0]
    z = l2n(a * zg + (1.0 - a) * zh)
    zg_ref[...] = zg
    zh_ref[...] = zh
    z_ref[...] = z
    lg_ref[...] = jnp.dot(z, wc_ref[...], preferred_element_type=jnp.float32) + bc_ref[...]


def _final(agg2, b2, bh2, alpha, wc, bc):
    nb = N // _RB_TC
    return pl.pallas_call(
        _final_body,
        grid=(nb,),
        in_specs=[
            pl.BlockSpec((_RB_TC, D), lambda i: (i, 0)),
            pl.BlockSpec((_RB_TC, D), lambda i: (i + N // _RB_TC, 0)),
            pl.BlockSpec((1, D), lambda i: (0, 0)),
            pl.BlockSpec((1, D), lambda i: (0, 0)),
            pl.BlockSpec(memory_space=pltpu.SMEM),
            pl.BlockSpec((D, NCLS), lambda i: (0, 0)),
            pl.BlockSpec((1, NCLS), lambda i: (0, 0)),
        ],
        out_specs=[
            pl.BlockSpec((_RB_TC, D), lambda i: (i, 0)),
            pl.BlockSpec((_RB_TC, D), lambda i: (i, 0)),
            pl.BlockSpec((_RB_TC, D), lambda i: (i, 0)),
            pl.BlockSpec((_RB_TC, NCLS), lambda i: (i, 0)),
        ],
        out_shape=[
            jax.ShapeDtypeStruct((N, D), jnp.float32),
            jax.ShapeDtypeStruct((N, D), jnp.float32),
            jax.ShapeDtypeStruct((N, D), jnp.float32),
            jax.ShapeDtypeStruct((N, NCLS), jnp.float32),
        ],
    )(agg2, agg2, b2, bh2, alpha, wc, bc)


@functools.lru_cache(maxsize=1)
def _make_sc_scatter():
    mesh = plsc.VectorSubcoreMesh(core_axis_name="c", subcore_axis_name="s")

    @functools.partial(
        pl.kernel,
        out_type=jax.ShapeDtypeStruct((2 * N, D), jnp.float32),
        mesh=mesh,
        scratch_types=[
            pltpu.VMEM_SHARED((N, D), jnp.float32),
            pltpu.VMEM((2, K), jnp.int32),
            pltpu.VMEM((K,), jnp.int32),
            pltpu.VMEM((K,), jnp.float32),
            pltpu.VMEM((K, D), jnp.float32),
            pltpu.VMEM((RB, D), jnp.float32),
        ],
    )
    def sc_scatter(table, packed, ew, out, acc, packb, adjb, ewb, gb, wb):
        cid = lax.axis_index("c")
        sid = lax.axis_index("s")

        def zrow(r, carry):
            for j in range(D // L):
                wb[r, pl.ds(j * L, L)] = jnp.zeros((L,), jnp.float32)
            return carry
        lax.fori_loop(0, RB, zrow, 0)
        for k in range(NFULL):
            pltpu.sync_copy(wb, acc.at[pl.ds((sid + k * NS) * RB, RB)])

        @pl.when(sid < NTAIL)
        def _zero_tail():
            pltpu.sync_copy(wb, acc.at[pl.ds((sid + NFULL * NS) * RB, RB)])
        plsc.subcore_barrier()

        base = sid * EPT
        branch_off = cid * N

        def chunk(c, carry):
            b = base + c * K
            pltpu.sync_copy(packed.at[:, pl.ds(b, K)], packb)
            pltpu.sync_copy(ew.at[pl.ds(b, K)], ewb)
            for j in range(K // L):
                adjb[pl.ds(j * L, L)] = packb[0, pl.ds(j * L, L)] + branch_off
            pltpu.sync_copy(table.at[adjb], gb)

            def group(g, icarry):
                wv = ewb[pl.ds(g * L, L)]
                for i in range(L):
                    w = wv[i]
                    e = g * L + i
                    for j in range(D // L):
                        gb[e, pl.ds(j * L, L)] = gb[e, pl.ds(j * L, L)] * w
                return icarry
            lax.fori_loop(0, K // L, group, 0)

            pltpu.sync_copy(gb, acc.at[packb.at[1]], add=True)
            return carry
        lax.fori_loop(0, CH, chunk, 0)
        plsc.subcore_barrier()

        for k in range(NFULL):
            r0 = (sid + k * NS) * RB
            pltpu.sync_copy(acc.at[pl.ds(r0, RB)], wb)
            pltpu.sync_copy(wb, out.at[pl.ds(cid * N + r0, RB)])

        @pl.when(sid < NTAIL)
        def _wb_tail():
            r0 = (sid + NFULL * NS) * RB
            pltpu.sync_copy(acc.at[pl.ds(r0, RB)], wb)
            pltpu.sync_copy(wb, out.at[pl.ds(cid * N + r0, RB)])

    return sc_scatter


def kernel(x, edge_index, edge_weight, W1, b1, W2, b2, Wh1, bh1, Wh2, bh2,
           alpha, Wc, bc):
    src = edge_index[0].astype(jnp.int32)
    dst = edge_index[1].astype(jnp.int32)
    pad = EP - E
    packed = jnp.stack([jnp.pad(src, (0, pad)), jnp.pad(dst, (0, pad))])
    ewp = jnp.pad(edge_weight.astype(jnp.float32), (0, pad))

    w1s = jnp.stack([W1, Wh1])
    w2s = jnp.stack([W2, Wh2])
    b1s = jnp.stack([b1, bh1]).reshape(2, 1, D)

    sc_scatter = _make_sc_scatter()
    table1 = _dense_first(x, w1s)
    agg1 = sc_scatter(table1, packed, ewp)
    table2 = _dense_mid(agg1, b1s, w2s)
    agg2 = sc_scatter(table2, packed, ewp)
    z_gcn, z_hgcn, z, logits = _final(
        agg2, b2.reshape(1, D), bh2.reshape(1, D),
        alpha.reshape(1, 1), Wc, bc.reshape(1, NCLS))
    return (z_gcn, z_hgcn, z, logits)


# speedup vs baseline: 3.4178x; 1.3138x over previous
import functools

import jax
import jax.numpy as jnp
from jax import lax
from jax.experimental import pallas as pl
from jax.experimental.pallas import tpu as pltpu
from jax.experimental.pallas import tpu_sc as plsc

N = 10000
E = 320000
D = 128
NCLS = 40
NC = 2
NS = 16
L = 16

EPT = 20480
EP = NS * EPT
K = 128
CH = EPT // K
CHP = CH // 2
RB = 80
NBLK = N // RB
NFULL = NBLK // NS
NTAIL = NBLK - NFULL * NS


def _mm_body(x_ref, w_ref, o_ref):
    o_ref[...] = jnp.dot(x_ref[...], w_ref[0], preferred_element_type=jnp.float32)


def _mm_relu_body(x_ref, b_ref, w_ref, o_ref):
    h = jnp.maximum(x_ref[...] + b_ref[0], 0.0)
    o_ref[...] = jnp.dot(h, w_ref[0], preferred_element_type=jnp.float32)


_RB_TC = 1000


def _dense_first(x, w_stacked):
    return pl.pallas_call(
        _mm_body,
        grid=(2 * N // _RB_TC,),
        in_specs=[
            pl.BlockSpec((_RB_TC, D), lambda i: (i % (N // _RB_TC), 0)),
            pl.BlockSpec((1, D, D), lambda i: (i // (N // _RB_TC), 0, 0)),
        ],
        out_specs=pl.BlockSpec((_RB_TC, D), lambda i: (i, 0)),
        out_shape=jax.ShapeDtypeStruct((2 * N, D), jnp.float32),
    )(x, w_stacked)


def _dense_mid(agg, b_stacked, w_stacked):
    return pl.pallas_call(
        _mm_relu_body,
        grid=(2 * N // _RB_TC,),
        in_specs=[
            pl.BlockSpec((_RB_TC, D), lambda i: (i, 0)),
            pl.BlockSpec((1, 1, D), lambda i: (i // (N // _RB_TC), 0, 0)),
            pl.BlockSpec((1, D, D), lambda i: (i // (N // _RB_TC), 0, 0)),
        ],
        out_specs=pl.BlockSpec((_RB_TC, D), lambda i: (i, 0)),
        out_shape=jax.ShapeDtypeStruct((2 * N, D), jnp.float32),
    )(agg, b_stacked, w_stacked)


def _final_body(g_ref, h_ref, b2_ref, bh2_ref, alpha_ref, wc_ref, bc_ref,
                zg_ref, zh_ref, z_ref, lg_ref):
    def l2n(v):
        nrm = jnp.sqrt(jnp.sum(v * v, axis=1, keepdims=True))
        return v / jnp.maximum(nrm, 1e-12)

    zg = l2n(g_ref[...] + b2_ref[...])
    zh = l2n(h_ref[...] + bh2_ref[...])
    a = alpha_ref[0, 0]
    z = l2n(a * zg + (1.0 - a) * zh)
    zg_ref[...] = zg
    zh_ref[...] = zh
    z_ref[...] = z
    lg_ref[...] = jnp.dot(z, wc_ref[...], preferred_element_type=jnp.float32) + bc_ref[...]


def _final(agg2, b2, bh2, alpha, wc, bc):
    nb = N // _RB_TC
    return pl.pallas_call(
        _final_body,
        grid=(nb,),
        in_specs=[
            pl.BlockSpec((_RB_TC, D), lambda i: (i, 0)),
            pl.BlockSpec((_RB_TC, D), lambda i: (i + N // _RB_TC, 0)),
            pl.BlockSpec((1, D), lambda i: (0, 0)),
            pl.BlockSpec((1, D), lambda i: (0, 0)),
            pl.BlockSpec(memory_space=pltpu.SMEM),
            pl.BlockSpec((D, NCLS), lambda i: (0, 0)),
            pl.BlockSpec((1, NCLS), lambda i: (0, 0)),
        ],
        out_specs=[
            pl.BlockSpec((_RB_TC, D), lambda i: (i, 0)),
            pl.BlockSpec((_RB_TC, D), lambda i: (i, 0)),
            pl.BlockSpec((_RB_TC, D), lambda i: (i, 0)),
            pl.BlockSpec((_RB_TC, NCLS), lambda i: (i, 0)),
        ],
        out_shape=[
            jax.ShapeDtypeStruct((N, D), jnp.float32),
            jax.ShapeDtypeStruct((N, D), jnp.float32),
            jax.ShapeDtypeStruct((N, D), jnp.float32),
            jax.ShapeDtypeStruct((N, NCLS), jnp.float32),
        ],
    )(agg2, agg2, b2, bh2, alpha, wc, bc)


@functools.lru_cache(maxsize=1)
def _make_sc_scatter():
    mesh = plsc.VectorSubcoreMesh(core_axis_name="c", subcore_axis_name="s")

    @functools.partial(
        pl.kernel,
        out_type=jax.ShapeDtypeStruct((2 * N, D), jnp.float32),
        mesh=mesh,
        scratch_types=[
            pltpu.VMEM_SHARED((N, D), jnp.float32),
            pltpu.VMEM((2, K), jnp.int32),
            pltpu.VMEM((2, K), jnp.int32),
            pltpu.VMEM((K,), jnp.float32),
            pltpu.VMEM((K,), jnp.float32),
            pltpu.VMEM((K,), jnp.int32),
            pltpu.VMEM((K,), jnp.int32),
            pltpu.VMEM((K,), jnp.int32),
            pltpu.VMEM((K,), jnp.int32),
            pltpu.VMEM((K, D), jnp.float32),
            pltpu.VMEM((K, D), jnp.float32),
            pltpu.VMEM((RB, D), jnp.float32),
            pltpu.SemaphoreType.DMA,
            pltpu.SemaphoreType.DMA,
            pltpu.SemaphoreType.DMA,
            pltpu.SemaphoreType.DMA,
            pltpu.SemaphoreType.DMA,
            pltpu.SemaphoreType.DMA,
        ],
    )
    def sc_scatter(table, packed, ew, out, acc,
                   pk0, pk1, ewb0, ewb1, adj0, adj1, db0, db1, gb0, gb1, wb,
                   sg0, sg1, ss0, ss1, sm0, sm1):
        cid = lax.axis_index("c")
        sid = lax.axis_index("s")

        def zrow(r, carry):
            for j in range(D // L):
                wb[r, pl.ds(j * L, L)] = jnp.zeros((L,), jnp.float32)
            return carry
        lax.fori_loop(0, RB, zrow, 0)
        for k in range(NFULL):
            pltpu.sync_copy(wb, acc.at[pl.ds((sid + k * NS) * RB, RB)])

        @pl.when(sid < NTAIL)
        def _zero_tail():
            pltpu.sync_copy(wb, acc.at[pl.ds((sid + NFULL * NS) * RB, RB)])
        plsc.subcore_barrier()

        base = sid * EPT
        branch_off = cid * N

        def meta_slice(c):
            return packed.at[:, pl.ds(base + c * K, K)]

        def ew_slice(c):
            return ew.at[pl.ds(base + c * K, K)]

        def issue_meta(c, pk, ewb, sem):
            pltpu.async_copy(meta_slice(c), pk, sem)
            pltpu.async_copy(ew_slice(c), ewb, sem)

        def wait_meta(c, pk, ewb, sem):
            pltpu.make_async_copy(meta_slice(c), pk, sem).wait()
            pltpu.make_async_copy(ew_slice(c), ewb, sem).wait()

        def compute_adj(pk, adj):
            for j in range(K // L):
                adj[pl.ds(j * L, L)] = pk[0, pl.ds(j * L, L)] + branch_off

        def multiply(ewb, gb):
            def group(g, icarry):
                wv = ewb[pl.ds(g * L, L)]
                for i in range(L):
                    w = wv[i]
                    e = g * L + i
                    for j in range(D // L):
                        gb[e, pl.ds(j * L, L)] = gb[e, pl.ds(j * L, L)] * w
                return icarry
            lax.fori_loop(0, K // L, group, 0)

        def issue_scatter(pk, db, gb, sem):
            for j in range(K // L):
                db[pl.ds(j * L, L)] = pk[1, pl.ds(j * L, L)]
            pltpu.async_copy(gb, acc.at[db], sem, add=True)

        pltpu.sync_copy(meta_slice(0), pk0)
        pltpu.sync_copy(ew_slice(0), ewb0)
        compute_adj(pk0, adj0)
        pltpu.async_copy(table.at[adj0], gb0, sg0)
        issue_meta(1, pk1, ewb1, sm1)

        def pair(cp, carry):
            c0 = 2 * cp
            pltpu.make_async_copy(table.at[adj0], gb0, sg0).wait()
            multiply(ewb0, gb0)
            issue_scatter(pk0, db0, gb0, ss0)

            @pl.when(cp < CHP - 1)
            def _meta_a():
                issue_meta(c0 + 2, pk0, ewb0, sm0)
            wait_meta(c0 + 1, pk1, ewb1, sm1)
            compute_adj(pk1, adj1)

            @pl.when(cp > 0)
            def _drain_s1():
                pltpu.make_async_copy(gb1, acc.at[db1], ss1).wait()
            pltpu.async_copy(table.at[adj1], gb1, sg1)

            pltpu.make_async_copy(table.at[adj1], gb1, sg1).wait()
            multiply(ewb1, gb1)
            issue_scatter(pk1, db1, gb1, ss1)

            @pl.when(cp < CHP - 1)
            def _meta_b():
                issue_meta(c0 + 3, pk1, ewb1, sm1)

            @pl.when(cp < CHP - 1)
            def _adj_a():
                wait_meta(c0 + 2, pk0, ewb0, sm0)
                compute_adj(pk0, adj0)
            pltpu.make_async_copy(gb0, acc.at[db0], ss0).wait()

            @pl.when(cp < CHP - 1)
            def _gather_a():
                pltpu.async_copy(table.at[adj0], gb0, sg0)
            return carry
        lax.fori_loop(0, CHP, pair, 0)
        pltpu.make_async_copy(gb1, acc.at[db1], ss1).wait()
        plsc.subcore_barrier()

        for k in range(NFULL):
            r0 = (sid + k * NS) * RB
            pltpu.sync_copy(acc.at[pl.ds(r0, RB)], wb)
            pltpu.sync_copy(wb, out.at[pl.ds(cid * N + r0, RB)])

        @pl.when(sid < NTAIL)
        def _wb_tail():
            r0 = (sid + NFULL * NS) * RB
            pltpu.sync_copy(acc.at[pl.ds(r0, RB)], wb)
            pltpu.sync_copy(wb, out.at[pl.ds(cid * N + r0, RB)])

    return sc_scatter


def kernel(x, edge_index, edge_weight, W1, b1, W2, b2, Wh1, bh1, Wh2, bh2,
           alpha, Wc, bc):
    src = edge_index[0].astype(jnp.int32)
    dst = edge_index[1].astype(jnp.int32)
    pad = EP - E
    packed = jnp.stack([jnp.pad(src, (0, pad)), jnp.pad(dst, (0, pad))])
    ewp = jnp.pad(edge_weight.astype(jnp.float32), (0, pad))

    w1s = jnp.stack([W1, Wh1])
    w2s = jnp.stack([W2, Wh2])
    b1s = jnp.stack([b1, bh1]).reshape(2, 1, D)

    sc_scatter = _make_sc_scatter()
    table1 = _dense_first(x, w1s)
    agg1 = sc_scatter(table1, packed, ewp)
    table2 = _dense_mid(agg1, b1s, w2s)
    agg2 = sc_scatter(table2, packed, ewp)
    z_gcn, z_hgcn, z, logits = _final(
        agg2, b2.reshape(1, D), bh2.reshape(1, D),
        alpha.reshape(1, 1), Wc, bc.reshape(1, NCLS))
    return (z_gcn, z_hgcn, z, logits)
